# baseline (speedup 1.0000x reference)
import functools

import jax
import jax.numpy as jnp
import numpy as np
from jax import lax
from jax.experimental import pallas as pl
from jax.experimental.pallas import tpu as pltpu
from jax.experimental.pallas import tpu_sc as plsc

_N = 10000
_K = 32
_D = 128
_H = 128
_NW = 32
_ROWS_PER_W = 320
_N_PAD = _NW * _ROWS_PER_W
_G = 4
_CHUNKS = _ROWS_PER_W // _G
_CG = _D // 32

_PERM = np.concatenate(
    [np.concatenate([np.arange(16) + 16 * cg, np.arange(16) + 64 + 16 * cg])
     for cg in range(_CG)])


def _sc_gather_sum(vf_i32, idx2):
    mesh = plsc.VectorSubcoreMesh(core_axis_name="c", subcore_axis_name="s")

    @functools.partial(
        pl.kernel,
        out_type=jax.ShapeDtypeStruct((_N, _D), jnp.float32),
        mesh=mesh,
        compiler_params=pltpu.CompilerParams(use_tc_tiling_on_sc=False),
        scratch_types=[
            pltpu.VMEM((_ROWS_PER_W, _K), jnp.int32),
            pltpu.VMEM((_G * _K, _D // 2), jnp.int32),
            pltpu.VMEM((_G * _K, _D // 2), jnp.int32),
            pltpu.VMEM((_ROWS_PER_W, _D), jnp.float32),
            pltpu.VMEM_SHARED((_N, _D // 2), jnp.int32),
            pltpu.SemaphoreType.DMA,
            pltpu.SemaphoreType.DMA,
        ],
    )
    def gather_sum(vf_hbm, idx_hbm, out_hbm, idx_v, rows_a, rows_b, out_v,
                   table_sp, sem_a, sem_b):
        wid = lax.axis_index("s") * 2 + lax.axis_index("c")
        sid = lax.axis_index("s")
        @pl.when(sid < 15)
        def _():
            pltpu.sync_copy(vf_hbm.at[pl.ds(sid * 624, 624)],
                            table_sp.at[pl.ds(sid * 624, 624)])

        @pl.when(sid == 15)
        def _():
            pltpu.sync_copy(vf_hbm.at[pl.ds(9360, 640)],
                            table_sp.at[pl.ds(9360, 640)])

        last_rows = _N - (_NW - 1) * _ROWS_PER_W

        @pl.when(wid < _NW - 1)
        def _():
            pltpu.sync_copy(
                idx_hbm.at[pl.ds(wid * _ROWS_PER_W, _ROWS_PER_W)], idx_v)

        @pl.when(wid == _NW - 1)
        def _():
            pltpu.sync_copy(
                idx_hbm.at[pl.ds((_NW - 1) * _ROWS_PER_W, last_rows)],
                idx_v.at[pl.ds(0, last_rows)])

        plsc.subcore_barrier()

        def start(g, rows, sem):
            for n in range(_G):
                pltpu.async_copy(table_sp.at[idx_v.at[g * _G + n]],
                                 rows.at[pl.ds(n * _K, _K)], sem)

        def wait(rows, sem):
            for n in range(_G):
                pltpu.make_async_copy(table_sp.at[idx_v.at[0]],
                                      rows.at[pl.ds(n * _K, _K)], sem).wait()

        def widen(rows, r, cg):
            w = rows[r, pl.ds(cg * 16, 16)]
            lo = lax.bitcast_convert_type(
                lax.shift_left(w, 16), jnp.float32)
            hi = lax.bitcast_convert_type(
                lax.bitwise_and(w, jnp.int32(-65536)), jnp.float32)
            return lo, hi

        def accum(rows, out_base):
            for n in range(_G):
                base = n * _K

                def tree4(r0):
                    acc = []
                    for cg in range(_CG):
                        a0, b0 = widen(rows, r0, cg)
                        a1, b1 = widen(rows, r0 + 1, cg)
                        a2, b2 = widen(rows, r0 + 2, cg)
                        a3, b3 = widen(rows, r0 + 3, cg)
                        acc.append((a0 + a1) + (a2 + a3))
                        acc.append((b0 + b1) + (b2 + b3))
                    return tuple(acc)

                parts = [tree4(base + q * 4) for q in range(_K // 4)]
                while len(parts) > 1:
                    parts = [
                        tuple(x + y for x, y in zip(parts[i], parts[i + 1]))
                        for i in range(0, len(parts), 2)]
                acc = parts[0]
                row = out_base + n
                for cg in range(_CG):
                    out_v[row, pl.ds(cg * 32, 16)] = acc[2 * cg]
                    out_v[row, pl.ds(cg * 32 + 16, 16)] = acc[2 * cg + 1]

        pairs = _CHUNKS // 2
        last_valid_full = _N - (_NW - 1) * _ROWS_PER_W
        pairs_w = jnp.where(wid == _NW - 1,
                            last_valid_full // (2 * _G), pairs)
        start(0, rows_a, sem_a)

        def pair_body(t, carry):
            g0 = 2 * t
            start(g0 + 1, rows_b, sem_b)
            wait(rows_a, sem_a)
            accum(rows_a, g0 * _G)

            @pl.when(t < pairs_w - 1)
            def _():
                start(g0 + 2, rows_a, sem_a)

            wait(rows_b, sem_b)
            accum(rows_b, (g0 + 1) * _G)
            return carry

        lax.fori_loop(0, pairs_w, pair_body, 0)

        @pl.when(wid < _NW - 1)
        def _():
            pltpu.sync_copy(
                out_v, out_hbm.at[pl.ds(wid * _ROWS_PER_W, _ROWS_PER_W)])

        last_valid = _N - (_NW - 1) * _ROWS_PER_W
        @pl.when(wid == _NW - 1)
        def _():
            pltpu.sync_copy(
                out_v.at[pl.ds(0, last_valid)],
                out_hbm.at[pl.ds((_NW - 1) * _ROWS_PER_W, last_valid)])

    return gather_sum(vf_i32, idx2)


def _tc_update(agg, vf, vl, W_p, B_w):
    R = 2000

    def body(agg_ref, vf_ref, vl_ref, w_ref, b_ref, out_ref):
        vlf = vl_ref[...].astype(jnp.float32)
        vlf = jnp.where(vlf == 0.0, 1.0, vlf)
        x = agg_ref[...] / vlf
        y = jnp.dot(x, w_ref[...], preferred_element_type=jnp.float32)
        y = y + jnp.dot(vf_ref[...], b_ref[...], preferred_element_type=jnp.float32)
        out_ref[...] = jnp.maximum(y, 0.0)

    return pl.pallas_call(
        body,
        grid=(_N // R,),
        in_specs=[
            pl.BlockSpec((R, _D), lambda i: (i, 0)),
            pl.BlockSpec((R, _D), lambda i: (i, 0)),
            pl.BlockSpec((R, 1), lambda i: (i, 0)),
            pl.BlockSpec((_D, _H), lambda i: (0, 0)),
            pl.BlockSpec((_D, _H), lambda i: (0, 0)),
        ],
        out_specs=pl.BlockSpec((R, _H), lambda i: (i, 0)),
        out_shape=jax.ShapeDtypeStruct((_N, _H), jnp.float32),
    )(agg, vf, vl, W_p, B_w)


def kernel(vertex_feat, neighbors_idx, valid_lens, W, B_w):
    vf = vertex_feat[0]
    b16 = lax.bitcast_convert_type(vf.astype(jnp.bfloat16), jnp.uint16)
    lo = b16[:, :_D // 2].astype(jnp.uint32)
    hi = b16[:, _D // 2:].astype(jnp.uint32)
    vf_i32 = lax.bitcast_convert_type(
        lo | (hi << jnp.uint32(16)), jnp.int32)
    agg = _sc_gather_sum(vf_i32, neighbors_idx[0])
    W_p = W[_PERM, :]
    out = _tc_update(agg, vf, valid_lens[0][:, None], W_p, B_w)
    return out[None]
